# baseline (speedup 1.0000x reference)
import functools

import jax
import jax.numpy as jnp
from jax.experimental import pallas as pl
from jax.experimental.pallas import tpu as pltpu
from jax.experimental.pallas import tpu_sc as plsc

B = 4096
S = 200
D = 128
TOKS = B * S
GATHER_W = 128
RB = 6400
NSTEP = TOKS // RB


def _sc_gather(tok_table, x_flat, n_rows):
    mesh = plsc.VectorSubcoreMesh(core_axis_name="c", subcore_axis_name="s")
    num_windows = n_rows // GATHER_W

    @functools.partial(
        pl.kernel,
        out_type=jax.ShapeDtypeStruct((n_rows, D), jnp.float32),
        mesh=mesh,
    )
    def gather_kernel(tok_hbm, idx_hbm, out_hbm):
        def body(idx_vmem, out_vmem):
            pltpu.sync_copy(tok_hbm.at[idx_vmem.at[0]], out_vmem)

        pltpu.emit_pipeline(
            body,
            grid=(num_windows,),
            in_specs=[pl.BlockSpec((1, GATHER_W), index_map=lambda i: (0, i))],
            out_specs=[pl.BlockSpec((GATHER_W, D), index_map=lambda i: (i, 0))],
            core_axis_name=("c", "s"),
            dimension_semantics=(pltpu.PARALLEL,),
        )(idx_hbm, out_hbm)

    return gather_kernel(tok_table, x_flat.reshape(1, n_rows))


def _ln_compute(g, segb, pos, segd, gam, bet):
    h = g + pos + segb * segd
    ones = jnp.full((D, D), 1.0 / D, jnp.float32)
    dims = (((1,), (0,)), ((), ()))
    mu = jax.lax.dot_general(h, ones, dims)
    sq = jax.lax.dot_general(h * h, ones, dims)
    var = sq - mu * mu
    return (h - mu) * jax.lax.rsqrt(var + 1e-5) * gam + bet


def _ln_body(g_hbm, segf_hbm, pos_ref, segd_ref, gam_ref, bet_ref, o_hbm,
             g_v, s_v, o_v, in_sem, seg_sem, out_sem):
    i = pl.program_id(0)

    def in_copy(step, slot):
        return (
            pltpu.make_async_copy(g_hbm.at[pl.ds(step * RB, RB)],
                                  g_v.at[slot], in_sem.at[slot]),
            pltpu.make_async_copy(segf_hbm.at[pl.ds(step * RB, RB)],
                                  s_v.at[slot], seg_sem.at[slot]),
        )

    def out_copy(step, slot):
        return pltpu.make_async_copy(o_v.at[slot],
                                     o_hbm.at[pl.ds(step * RB, RB)],
                                     out_sem.at[slot])

    slot = jax.lax.rem(i, 2)
    nslot = jax.lax.rem(i + 1, 2)

    @pl.when(i == 0)
    def _():
        for c in in_copy(0, 0):
            c.start()

    @pl.when(i + 1 < NSTEP)
    def _():
        for c in in_copy(i + 1, nslot):
            c.start()

    for c in in_copy(i, slot):
        c.wait()

    @pl.when(i >= 2)
    def _():
        out_copy(i - 2, slot).wait()

    o_v[slot] = _ln_compute(g_v[slot], s_v[slot], pos_ref[...],
                            segd_ref[...], gam_ref[...], bet_ref[...])
    out_copy(i, slot).start()

    @pl.when(i == NSTEP - 1)
    def _():
        out_copy(i - 1, nslot).wait()
        out_copy(i, slot).wait()


def kernel(x, seg, tok_table, pos_table, seg_table, ln_gamma, ln_beta):
    x_flat = x.reshape(-1).astype(jnp.int32)
    segf = seg.astype(jnp.float32).reshape(TOKS, 1)
    posP = pos_table[:S] + seg_table[0][None, :]
    pos_tile = jnp.tile(posP, (RB // S, 1))
    segd = (seg_table[1] - seg_table[0]).reshape(1, D)
    gamma = ln_gamma.reshape(1, D)
    beta = ln_beta.reshape(1, D)

    gathered = _sc_gather(tok_table, x_flat, TOKS)
    out = pl.pallas_call(
        _ln_body,
        grid=(NSTEP,),
        in_specs=[
            pl.BlockSpec(memory_space=pl.ANY),
            pl.BlockSpec(memory_space=pl.ANY),
            pl.BlockSpec((RB, D), lambda i: (0, 0)),
            pl.BlockSpec((1, D), lambda i: (0, 0)),
            pl.BlockSpec((1, D), lambda i: (0, 0)),
            pl.BlockSpec((1, D), lambda i: (0, 0)),
        ],
        out_specs=pl.BlockSpec(memory_space=pl.ANY),
        out_shape=jax.ShapeDtypeStruct((TOKS, D), jnp.float32),
        scratch_shapes=[
            pltpu.VMEM((2, RB, D), jnp.float32),
            pltpu.VMEM((2, RB, 1), jnp.float32),
            pltpu.VMEM((2, RB, D), jnp.float32),
            pltpu.SemaphoreType.DMA((2,)),
            pltpu.SemaphoreType.DMA((2,)),
            pltpu.SemaphoreType.DMA((2,)),
        ],
    )(gathered, segf, pos_tile, segd, gamma, beta)
    return out.reshape(B, S, D)
